# baseline (speedup 1.0000x reference)
import functools

import jax
import jax.numpy as jnp
import numpy as np
from jax import lax
from jax.experimental import pallas as pl
from jax.experimental.pallas import tpu as pltpu
from jax.experimental.pallas import tpu_sc as plsc

N = 10000
D = 128
E = 320000
NUM_LAYERS = 3

NC = 2
NS = 16
NW = NC * NS
CHUNK = 128
NPAD = 10240
EPAD = 327680
CPW = EPAD // (NW * CHUNK)
CH = 64
NCH = EPAD // (NW * CH)
NBUF = 5
GRP = 8
RPT = NPAD // NS

_mesh = plsc.VectorSubcoreMesh(core_axis_name="c", subcore_axis_name="s")


@functools.partial(
    pl.kernel,
    out_type=jax.ShapeDtypeStruct((NC, NPAD), jnp.float32),
    mesh=_mesh,
    scratch_types=[
        pltpu.VMEM_SHARED((NPAD,), jnp.float32),
        pltpu.VMEM((CPW, CHUNK), jnp.int32),
        pltpu.VMEM((CHUNK,), jnp.float32),
        pltpu.VMEM((RPT,), jnp.float32),
        pltpu.SemaphoreType.DMA,
    ],
)
def _deg_kernel(cols_hbm, out_hbm, acc, cbuf, ones, zbuf, dsem):
    c = lax.axis_index("c")
    s = lax.axis_index("s")
    wid = c * NS + s

    @pl.loop(0, CHUNK // 16)
    def _(i):
        ones[pl.ds(i * 16, 16)] = jnp.ones((16,), jnp.float32)

    @pl.loop(0, RPT // 16)
    def _(i):
        zbuf[pl.ds(i * 16, 16)] = jnp.zeros((16,), jnp.float32)

    pltpu.sync_copy(zbuf, acc.at[pl.ds(s * RPT, RPT)])
    plsc.subcore_barrier()

    pltpu.sync_copy(cols_hbm.at[pl.ds(wid * CPW, CPW)], cbuf)

    pend = [
        pltpu.async_copy(ones, acc.at[cbuf.at[j]], dsem, add=True)
        for j in range(CPW)
    ]
    for h in pend:
        h.wait()

    plsc.subcore_barrier()
    pltpu.sync_copy(acc.at[pl.ds(s * RPT, RPT)], out_hbm.at[c, pl.ds(s * RPT, RPT)])


@functools.partial(
    pl.kernel,
    out_type=jax.ShapeDtypeStruct((NC, NPAD, D), jnp.float32),
    mesh=_mesh,
    scratch_types=[
        pltpu.VMEM_SHARED((NPAD, D), jnp.float32),
        pltpu.VMEM((2, GRP // 2, 2, CH), jnp.int32),
        pltpu.VMEM((2, GRP // 2, 2, CH), jnp.int32),
        pltpu.VMEM((NBUF, CH, D), jnp.float32),
        pltpu.SemaphoreType.DMA,
        pltpu.SemaphoreType.DMA,
        pltpu.SemaphoreType.DMA,
        pltpu.SemaphoreType.DMA,
        pltpu.SemaphoreType.DMA,
        pltpu.SemaphoreType.DMA,
        pltpu.SemaphoreType.DMA,
        pltpu.SemaphoreType.DMA,
        pltpu.SemaphoreType.DMA,
        pltpu.SemaphoreType.DMA,
        pltpu.SemaphoreType.DMA,
    ],
)
def _prop_kernel(u_hbm, rows_hbm, cols_hbm, out_hbm, acc, rbuf, cbuf, mbuf,
                 ga, gb, gc, gd, ge, sa, sb, sc, sd, se, isem):
    c = lax.axis_index("c")
    s = lax.axis_index("s")
    wid = c * NS + s
    gsem = (ga, gb, gc, gd, ge)
    ssem = (sa, sb, sc, sd, se)

    def stage(g, slot):
        sl = pl.ds((wid * NCH + g * GRP) // 2, GRP // 2)
        a = pltpu.async_copy(rows_hbm.at[sl], rbuf.at[slot], isem)
        b = pltpu.async_copy(cols_hbm.at[sl], cbuf.at[slot], isem)
        return (a, b)

    def gather(j, b):
        g, t = j // GRP, j % GRP
        return pltpu.async_copy(
            u_hbm.at[rbuf.at[g % 2, t // 2, t % 2]], mbuf.at[b], gsem[b]
        )

    pend_g = [None] * NBUF
    pend_s = [None] * NBUF
    st = stage(0, 0)

    @pl.loop(0, CH)
    def _(i):
        @pl.loop(0, D // 16)
        def _(k):
            mbuf[0, i, pl.ds(k * 16, 16)] = jnp.zeros((16,), jnp.float32)

    st[0].wait()
    st[1].wait()
    pend_stage = None
    pend_g[2] = gather(0, 2)
    pend_g[3] = gather(1, 3)
    pend_g[4] = gather(2, 4)

    @pl.loop(0, RPT // CH)
    def _(k):
        pltpu.sync_copy(mbuf.at[0], acc.at[pl.ds(s * RPT + k * CH, CH)])

    plsc.subcore_barrier()

    ngroups = NCH // GRP
    for g in range(ngroups):
        for t in range(GRP):
            if t == 2 and g + 1 < ngroups:
                pend_stage = stage(g + 1, (g + 1) % 2)
            if t == GRP - 4 and pend_stage is not None:
                pend_stage[0].wait()
                pend_stage[1].wait()
                pend_stage = None
            j = g * GRP + t
            b = (j + 2) % NBUF
            pend_g[b].wait()
            pend_s[b] = pltpu.async_copy(
                mbuf.at[b], acc.at[cbuf.at[g % 2, t // 2, t % 2]], ssem[b], add=True
            )
            jj = j + 3
            if jj < NCH:
                b2 = (jj + 2) % NBUF
                if pend_s[b2] is not None:
                    pend_s[b2].wait()
                    pend_s[b2] = None
                pend_g[b2] = gather(jj, b2)
    for b in range(NBUF):
        if pend_s[b] is not None:
            pend_s[b].wait()

    plsc.subcore_barrier()
    pltpu.sync_copy(
        acc.at[pl.ds(s * RPT, RPT)], out_hbm.at[c, pl.ds(s * RPT, RPT)]
    )


def _degnorm_body(degp_ref, dinv_ref):
    deg = degp_ref[0] + degp_ref[1]
    dinv_ref[...] = jnp.where(deg > 0.0, lax.rsqrt(deg), 0.0)


def _scale_body(dinv_ref, x_ref, u_ref):
    u_ref[...] = dinv_ref[...] * x_ref[...]


def _layer_body(p_ref, dinv_ref, y_ref, u_ref):
    y = dinv_ref[...] * (p_ref[0] + p_ref[1])
    y_ref[...] = y
    u_ref[...] = dinv_ref[...] * y


def _last_body(p_ref, dinv_ref, x_ref, y1_ref, y2_ref, out_ref):
    y3 = dinv_ref[...] * (p_ref[0] + p_ref[1])
    out_ref[...] = 0.25 * (x_ref[...] + y1_ref[...] + y2_ref[...] + y3)


def _pad_body(e_ref, rpad_ref, cpad_ref, c128_ref, r64_ref, c64_ref):
    r = jnp.concatenate([e_ref[0], rpad_ref[...]], axis=0)
    c = jnp.concatenate([e_ref[1], cpad_ref[...]], axis=0)
    c128_ref[...] = c
    r64_ref[:, 0, :] = r[:, :CH]
    r64_ref[:, 1, :] = r[:, CH:]
    c64_ref[:, 0, :] = c[:, :CH]
    c64_ref[:, 1, :] = c[:, CH:]


BLK = 1000
_row_spec = pl.BlockSpec((BLK, D), lambda i: (i, 0))
_p_spec = pl.BlockSpec((NC, BLK, D), lambda i: (0, i, 0))
_d_spec = pl.BlockSpec((BLK, 1), lambda i: (i, 0))


def kernel(emb_weight, edge_index):
    pi = np.arange(EPAD - E, dtype=np.int32)
    rpad = jnp.asarray((pi % N).reshape(-1, CHUNK))
    cpad = jnp.asarray((N + pi % (NPAD - N)).reshape(-1, CHUNK))
    cols2d, rows64, cols64 = pl.pallas_call(
        _pad_body,
        out_shape=(
            jax.ShapeDtypeStruct((EPAD // CHUNK, CHUNK), jnp.int32),
            jax.ShapeDtypeStruct((EPAD // CHUNK, 2, CH), jnp.int32),
            jax.ShapeDtypeStruct((EPAD // CHUNK, 2, CH), jnp.int32),
        ),
    )(edge_index.astype(jnp.int32).reshape(2, E // CHUNK, CHUNK), rpad, cpad)

    degp = _deg_kernel(cols2d)

    dinv2d = pl.pallas_call(
        _degnorm_body,
        out_shape=jax.ShapeDtypeStruct((NPAD // 128, 128), jnp.float32),
    )(degp.reshape(NC, NPAD // 128, 128))
    dinv_col = dinv2d.reshape(NPAD, 1)[:N]

    u = pl.pallas_call(
        _scale_body,
        grid=(N // BLK,),
        in_specs=[_d_spec, _row_spec],
        out_specs=_row_spec,
        out_shape=jax.ShapeDtypeStruct((N, D), jnp.float32),
    )(dinv_col, emb_weight)

    ys = []
    for _ in range(NUM_LAYERS - 1):
        p = _prop_kernel(u, rows64, cols64)
        y, u = pl.pallas_call(
            _layer_body,
            grid=(N // BLK,),
            in_specs=[_p_spec, _d_spec],
            out_specs=(_row_spec, _row_spec),
            out_shape=(
                jax.ShapeDtypeStruct((N, D), jnp.float32),
                jax.ShapeDtypeStruct((N, D), jnp.float32),
            ),
        )(p, dinv_col)
        ys.append(y)

    p = _prop_kernel(u, rows64, cols64)
    out = pl.pallas_call(
        _last_body,
        grid=(N // BLK,),
        in_specs=[_p_spec, _d_spec, _row_spec, _row_spec, _row_spec],
        out_specs=_row_spec,
        out_shape=jax.ShapeDtypeStruct((N, D), jnp.float32),
    )(p, dinv_col, emb_weight, ys[0], ys[1])
    return out


# speedup vs baseline: 1.0317x; 1.0317x over previous
import functools

import jax
import jax.numpy as jnp
import numpy as np
from jax import lax
from jax.experimental import pallas as pl
from jax.experimental.pallas import tpu as pltpu
from jax.experimental.pallas import tpu_sc as plsc

N = 10000
D = 128
E = 320000
NUM_LAYERS = 3

NC = 2
NS = 16
NW = NC * NS
CHUNK = 128
NPAD = 10240
EPAD = 327680
CPW = EPAD // (NW * CHUNK)
CH = 64
NCH = EPAD // (NW * CH)
NBUF = 5
GRP = 8
RPT = NPAD // NS

_mesh = plsc.VectorSubcoreMesh(core_axis_name="c", subcore_axis_name="s")


@functools.partial(
    pl.kernel,
    out_type=jax.ShapeDtypeStruct((NC, NPAD), jnp.float32),
    mesh=_mesh,
    scratch_types=[
        pltpu.VMEM_SHARED((NPAD,), jnp.float32),
        pltpu.VMEM((CPW, CHUNK), jnp.int32),
        pltpu.VMEM((CHUNK,), jnp.float32),
        pltpu.VMEM((RPT,), jnp.float32),
        pltpu.SemaphoreType.DMA,
    ],
)
def _deg_kernel(cols_hbm, out_hbm, acc, cbuf, ones, zbuf, dsem):
    c = lax.axis_index("c")
    s = lax.axis_index("s")
    wid = c * NS + s

    @pl.loop(0, CHUNK // 16)
    def _(i):
        ones[pl.ds(i * 16, 16)] = jnp.ones((16,), jnp.float32)

    @pl.loop(0, RPT // 16)
    def _(i):
        zbuf[pl.ds(i * 16, 16)] = jnp.zeros((16,), jnp.float32)

    pltpu.sync_copy(zbuf, acc.at[pl.ds(s * RPT, RPT)])
    plsc.subcore_barrier()

    pltpu.sync_copy(cols_hbm.at[pl.ds(wid * CPW, CPW)], cbuf)

    pend = [
        pltpu.async_copy(ones, acc.at[cbuf.at[j]], dsem, add=True)
        for j in range(CPW)
    ]
    for h in pend:
        h.wait()

    plsc.subcore_barrier()
    pltpu.sync_copy(acc.at[pl.ds(s * RPT, RPT)], out_hbm.at[c, pl.ds(s * RPT, RPT)])


@functools.partial(
    pl.kernel,
    out_type=jax.ShapeDtypeStruct((NC, NPAD, D), jnp.float32),
    mesh=_mesh,
    scratch_types=[
        pltpu.VMEM_SHARED((NPAD, D), jnp.float32),
        pltpu.VMEM((2, GRP, CH), jnp.int32),
        pltpu.VMEM((2, GRP, CH), jnp.int32),
        pltpu.VMEM((NBUF, CH, D), jnp.float32),
        pltpu.SemaphoreType.DMA,
        pltpu.SemaphoreType.DMA,
        pltpu.SemaphoreType.DMA,
        pltpu.SemaphoreType.DMA,
        pltpu.SemaphoreType.DMA,
        pltpu.SemaphoreType.DMA,
        pltpu.SemaphoreType.DMA,
        pltpu.SemaphoreType.DMA,
        pltpu.SemaphoreType.DMA,
        pltpu.SemaphoreType.DMA,
        pltpu.SemaphoreType.DMA,
    ],
)
def _prop_kernel(u_hbm, rows_hbm, cols_hbm, out_hbm, acc, rbuf, cbuf, mbuf,
                 ga, gb, gc, gd, ge, sa, sb, sc, sd, se, isem):
    c = lax.axis_index("c")
    s = lax.axis_index("s")
    wid = c * NS + s
    gsem = (ga, gb, gc, gd, ge)
    ssem = (sa, sb, sc, sd, se)

    def stage(g, slot):
        sl = pl.ds(wid * NCH + g * GRP, GRP)
        a = pltpu.async_copy(rows_hbm.at[sl], rbuf.at[slot], isem)
        b = pltpu.async_copy(cols_hbm.at[sl], cbuf.at[slot], isem)
        return (a, b)

    def gather(j, b):
        g, t = j // GRP, j % GRP
        return pltpu.async_copy(
            u_hbm.at[rbuf.at[g % 2, t]], mbuf.at[b], gsem[b]
        )

    pend_g = [None] * NBUF
    pend_s = [None] * NBUF
    st = stage(0, 0)

    @pl.loop(0, CH)
    def _(i):
        @pl.loop(0, D // 16)
        def _(k):
            mbuf[0, i, pl.ds(k * 16, 16)] = jnp.zeros((16,), jnp.float32)

    st[0].wait()
    st[1].wait()
    pend_stage = None
    pend_g[2] = gather(0, 2)
    pend_g[3] = gather(1, 3)
    pend_g[4] = gather(2, 4)

    @pl.loop(0, RPT // CH)
    def _(k):
        pltpu.sync_copy(mbuf.at[0], acc.at[pl.ds(s * RPT + k * CH, CH)])

    plsc.subcore_barrier()

    ngroups = NCH // GRP
    for g in range(ngroups):
        for t in range(GRP):
            if t == 2 and g + 1 < ngroups:
                pend_stage = stage(g + 1, (g + 1) % 2)
            if t == GRP - 4 and pend_stage is not None:
                pend_stage[0].wait()
                pend_stage[1].wait()
                pend_stage = None
            j = g * GRP + t
            b = (j + 2) % NBUF
            pend_g[b].wait()
            pend_s[b] = pltpu.async_copy(
                mbuf.at[b], acc.at[cbuf.at[g % 2, t]], ssem[b], add=True
            )
            jj = j + 3
            if jj < NCH:
                b2 = (jj + 2) % NBUF
                if pend_s[b2] is not None:
                    pend_s[b2].wait()
                    pend_s[b2] = None
                pend_g[b2] = gather(jj, b2)
    for b in range(NBUF):
        if pend_s[b] is not None:
            pend_s[b].wait()

    plsc.subcore_barrier()
    pltpu.sync_copy(
        acc.at[pl.ds(s * RPT, RPT)], out_hbm.at[c, pl.ds(s * RPT, RPT)]
    )


def _degnorm_body(degp_ref, dinv_ref):
    deg = degp_ref[0] + degp_ref[1]
    dinv_ref[...] = jnp.where(deg > 0.0, lax.rsqrt(deg), 0.0)


def _scale_body(dinv_ref, x_ref, u_ref):
    u_ref[...] = dinv_ref[...] * x_ref[...]


def _layer_body(p_ref, dinv_ref, y_ref, u_ref):
    y = dinv_ref[...] * (p_ref[0] + p_ref[1])
    y_ref[...] = y
    u_ref[...] = dinv_ref[...] * y


def _last_body(p_ref, dinv_ref, x_ref, y1_ref, y2_ref, out_ref):
    y3 = dinv_ref[...] * (p_ref[0] + p_ref[1])
    out_ref[...] = 0.25 * (x_ref[...] + y1_ref[...] + y2_ref[...] + y3)


def _pad_body(e_ref, rpad_ref, cpad_ref, r_ref, c_ref):
    r_ref[: E // CHUNK] = e_ref[0]
    r_ref[E // CHUNK :] = rpad_ref[...]
    c_ref[: E // CHUNK] = e_ref[1]
    c_ref[E // CHUNK :] = cpad_ref[...]


BLK = 2000
_row_spec = pl.BlockSpec((BLK, D), lambda i: (i, 0))
_p_spec = pl.BlockSpec((NC, BLK, D), lambda i: (0, i, 0))
_d_spec = pl.BlockSpec((BLK, 1), lambda i: (i, 0))


def kernel(emb_weight, edge_index):
    pi = np.arange(EPAD - E, dtype=np.int32)
    rpad = jnp.asarray((pi % N).reshape(-1, CHUNK))
    cpad = jnp.asarray((N + pi % (NPAD - N)).reshape(-1, CHUNK))
    rows2d, cols2d = pl.pallas_call(
        _pad_body,
        out_shape=(
            jax.ShapeDtypeStruct((EPAD // CHUNK, CHUNK), jnp.int32),
            jax.ShapeDtypeStruct((EPAD // CHUNK, CHUNK), jnp.int32),
        ),
    )(edge_index.astype(jnp.int32).reshape(2, E // CHUNK, CHUNK), rpad, cpad)

    degp = _deg_kernel(cols2d)

    dinv2d = pl.pallas_call(
        _degnorm_body,
        out_shape=jax.ShapeDtypeStruct((NPAD // 128, 128), jnp.float32),
    )(degp.reshape(NC, NPAD // 128, 128))
    dinv_col = dinv2d.reshape(NPAD, 1)[:N]

    u = pl.pallas_call(
        _scale_body,
        grid=(N // BLK,),
        in_specs=[_d_spec, _row_spec],
        out_specs=_row_spec,
        out_shape=jax.ShapeDtypeStruct((N, D), jnp.float32),
    )(dinv_col, emb_weight)

    rows64 = rows2d.reshape(EPAD // CH, CH)
    cols64 = cols2d.reshape(EPAD // CH, CH)
    ys = []
    for _ in range(NUM_LAYERS - 1):
        p = _prop_kernel(u, rows64, cols64)
        y, u = pl.pallas_call(
            _layer_body,
            grid=(N // BLK,),
            in_specs=[_p_spec, _d_spec],
            out_specs=(_row_spec, _row_spec),
            out_shape=(
                jax.ShapeDtypeStruct((N, D), jnp.float32),
                jax.ShapeDtypeStruct((N, D), jnp.float32),
            ),
        )(p, dinv_col)
        ys.append(y)

    p = _prop_kernel(u, rows64, cols64)
    out = pl.pallas_call(
        _last_body,
        grid=(N // BLK,),
        in_specs=[_p_spec, _d_spec, _row_spec, _row_spec, _row_spec],
        out_specs=_row_spec,
        out_shape=jax.ShapeDtypeStruct((N, D), jnp.float32),
    )(p, dinv_col, emb_weight, ys[0], ys[1])
    return out
